# baseline (speedup 1.0000x reference)
import functools

import jax
import jax.numpy as jnp
from jax import lax
from jax.experimental import pallas as pl
from jax.experimental.pallas import tpu as pltpu
from jax.experimental.pallas import tpu_sc as plsc

_EPS = 1e-8


def _sim_z_block(x_ref, p_ref, a_ref, simT_ref, z_ref):
    xb = x_ref[:, :]
    p = p_ref[:, :]
    xnorm = jnp.sqrt(jnp.sum(xb * xb, axis=1, keepdims=True))
    pnorm = jnp.sqrt(jnp.sum(p * p, axis=1, keepdims=True))
    xn = xb / (xnorm + _EPS)
    pn = p / (pnorm + _EPS)
    sT = jax.lax.dot_general(
        pn, xn, (((1,), (1,)), ((), ())),
        preferred_element_type=jnp.float32)
    simT_ref[:, :] = jnp.abs(sT)
    z = jax.lax.dot_general(
        xb.astype(jnp.bfloat16), a_ref[:, :], (((1,), (1,)), ((), ())),
        preferred_element_type=jnp.float32)
    z_ref[:, :] = z.astype(jnp.bfloat16)


def _delta_block(z_ref, wT_ref, b_ref, o_ref, *, rank):
    wT = wT_ref[:, :]
    e = wT.shape[0]
    er = b_ref.shape[0]
    row_e = jax.lax.broadcasted_iota(jnp.int32, (e, er), 0)
    col_e = jax.lax.broadcasted_iota(jnp.int32, (e, er), 1) // rank
    exp_m = (row_e == col_e).astype(jnp.float32)
    w_exp = jax.lax.dot_general(
        wT, exp_m, (((0,), (0,)), ((), ())),
        precision=jax.lax.Precision.HIGHEST,
        preferred_element_type=jnp.float32)
    u = (z_ref[:, :].astype(jnp.float32) * w_exp).astype(jnp.bfloat16)
    o_ref[:, :] = jax.lax.dot_general(
        u, b_ref[:, :], (((1,), (0,)), ((), ())),
        preferred_element_type=jnp.float32)


def _make_sc_router(n, e, n_workers, tokens_per_worker):
    mesh = plsc.VectorSubcoreMesh(
        core_axis_name="c", subcore_axis_name="s",
        num_cores=2, num_subcores=16)

    @functools.partial(
        pl.kernel,
        out_type=jax.ShapeDtypeStruct((e, n), jnp.float32),
        mesh=mesh,
        scratch_types=[
            pltpu.VMEM((e, tokens_per_worker), jnp.float32),
            pltpu.VMEM((e, tokens_per_worker), jnp.float32),
        ],
    )
    def route(simT_hbm, wT_hbm, simv, wv):
        n_cores = n_workers // 16
        wid = lax.axis_index("s") * n_cores + lax.axis_index("c")
        base = wid * tokens_per_worker
        pltpu.sync_copy(simT_hbm.at[:, pl.ds(base, tokens_per_worker)], simv)
        for j in range(tokens_per_worker // 16):
            sl = pl.ds(j * 16, 16)
            rows = [simv[ei, sl] for ei in range(e)]
            m1 = rows[0]
            for ei in range(1, e):
                m1 = jnp.maximum(m1, rows[ei])
            idx1 = jnp.zeros((16,), jnp.int32)
            for ei in range(e - 1, -1, -1):
                idx1 = jnp.where(rows[ei] == m1, ei, idx1)
            neg = jnp.full((16,), -1.0, jnp.float32)
            masked = [jnp.where(idx1 == ei, neg, rows[ei]) for ei in range(e)]
            m2 = masked[0]
            for ei in range(1, e):
                m2 = jnp.maximum(m2, masked[ei])
            idx2 = jnp.zeros((16,), jnp.int32)
            for ei in range(e - 1, -1, -1):
                idx2 = jnp.where(masked[ei] == m2, ei, idx2)
            ex = jnp.exp(m2 - m1)
            c1 = 1.0 / (1.0 + ex)
            c2 = 1.0 - c1
            for ei in range(e):
                wv[ei, sl] = jnp.where(
                    idx1 == ei, c1, jnp.where(idx2 == ei, c2, 0.0))
        pltpu.sync_copy(wv, wT_hbm.at[:, pl.ds(base, tokens_per_worker)])

    return route


def kernel(x, lora_A, lora_B, prototypes, scaling):
    bsz, seq, d = x.shape
    e, r, _ = lora_A.shape
    n = bsz * seq
    flat_x = x.reshape(n, d)
    a_stack = lora_A.reshape(e * r, d).astype(jnp.bfloat16)
    b_stack = (lora_B.transpose(0, 2, 1).reshape(e * r, d)
               * jnp.float32(scaling)).astype(jnp.bfloat16)

    tn = 1024
    grid = (n // tn,)
    sim_t, z = pl.pallas_call(
        _sim_z_block,
        grid=grid,
        in_specs=[
            pl.BlockSpec((tn, d), lambda i: (i, 0)),
            pl.BlockSpec((e, d), lambda i: (0, 0)),
            pl.BlockSpec((e * r, d), lambda i: (0, 0)),
        ],
        out_specs=[
            pl.BlockSpec((e, tn), lambda i: (0, i)),
            pl.BlockSpec((tn, e * r), lambda i: (i, 0)),
        ],
        out_shape=[
            jax.ShapeDtypeStruct((e, n), jnp.float32),
            jax.ShapeDtypeStruct((n, e * r), jnp.bfloat16),
        ],
    )(flat_x, prototypes, a_stack)

    n_workers = 32
    w_t = _make_sc_router(n, e, n_workers, n // n_workers)(sim_t)

    out = pl.pallas_call(
        functools.partial(_delta_block, rank=r),
        grid=grid,
        in_specs=[
            pl.BlockSpec((tn, e * r), lambda i: (i, 0)),
            pl.BlockSpec((e, tn), lambda i: (0, i)),
            pl.BlockSpec((e * r, d), lambda i: (0, 0)),
        ],
        out_specs=pl.BlockSpec((tn, d), lambda i: (i, 0)),
        out_shape=jax.ShapeDtypeStruct((n, d), jnp.float32),
    )(z, w_t, b_stack)
    return out.reshape(bsz, seq, d)


# speedup vs baseline: 1.7273x; 1.7273x over previous
import functools

import jax
import jax.numpy as jnp
from jax.experimental import pallas as pl
from jax.experimental.pallas import tpu as pltpu

_EPS = 1e-8


def _fused_block(x_ref, p_ref, a_ref, b_ref, o_ref, *, rank):
    xb = x_ref[:, :]
    p = p_ref[:, :]
    tn = xb.shape[0]
    e = p.shape[0]
    er = a_ref.shape[0]

    xnorm = jnp.sqrt(jnp.sum(xb * xb, axis=1, keepdims=True))
    pnorm = jnp.sqrt(jnp.sum(p * p, axis=1, keepdims=True))
    xn = xb / (xnorm + _EPS)
    pn = p / (pnorm + _EPS)
    s = jax.lax.dot_general(
        xn, pn, (((1,), (1,)), ((), ())),
        preferred_element_type=jnp.float32)
    sim = jnp.abs(s)

    iota_e = jax.lax.broadcasted_iota(jnp.int32, (tn, e), 1)
    m1 = jnp.max(sim, axis=1, keepdims=True)
    idx1 = jnp.min(jnp.where(sim == m1, iota_e, e), axis=1, keepdims=True)
    masked = jnp.where(iota_e == idx1, -1.0, sim)
    m2 = jnp.max(masked, axis=1, keepdims=True)
    idx2 = jnp.min(jnp.where(masked == m2, iota_e, e), axis=1, keepdims=True)
    c1 = jax.nn.sigmoid(m1 - m2)
    c2 = jax.nn.sigmoid(m2 - m1)

    col_e = jax.lax.broadcasted_iota(jnp.int32, (1, er), 1) // rank
    w = (jnp.where(col_e == idx1, c1, 0.0)
         + jnp.where(col_e == idx2, c2, 0.0))

    z = jax.lax.dot_general(
        xb.astype(jnp.bfloat16), a_ref[:, :], (((1,), (1,)), ((), ())),
        preferred_element_type=jnp.float32)
    u = (z * w).astype(jnp.bfloat16)
    delta = jax.lax.dot_general(
        u, b_ref[:, :], (((1,), (0,)), ((), ())),
        preferred_element_type=jnp.float32)
    o_ref[:, :] = delta


def kernel(x, lora_A, lora_B, prototypes, scaling):
    bsz, seq, d = x.shape
    e, r, _ = lora_A.shape
    n = bsz * seq
    flat_x = x.reshape(n, d)
    a_stack = lora_A.reshape(e * r, d).astype(jnp.bfloat16)
    b_stack = (lora_B.transpose(0, 2, 1).reshape(e * r, d)
               * jnp.float32(scaling)).astype(jnp.bfloat16)

    tn = 1024
    grid = (n // tn,)
    out = pl.pallas_call(
        functools.partial(_fused_block, rank=r),
        grid=grid,
        in_specs=[
            pl.BlockSpec((tn, d), lambda i: (i, 0)),
            pl.BlockSpec((e, d), lambda i: (0, 0)),
            pl.BlockSpec((e * r, d), lambda i: (0, 0)),
            pl.BlockSpec((e * r, d), lambda i: (0, 0)),
        ],
        out_specs=pl.BlockSpec((tn, d), lambda i: (i, 0)),
        out_shape=jax.ShapeDtypeStruct((n, d), jnp.float32),
        compiler_params=pltpu.CompilerParams(
            dimension_semantics=("parallel",)),
    )(flat_x, prototypes, a_stack, b_stack)
    return out.reshape(bsz, seq, d)
